# Initial kernel scaffold; baseline (speedup 1.0000x reference)
import jax
import jax.numpy as jnp
from jax.experimental import pallas as pl


def kernel(eachimg_selected_box_nums, box_features, positions):
    raise NotImplementedError("write your pallas kernel here")


# speedup vs baseline: 5.8253x; 5.8253x over previous
import functools

import jax
import jax.numpy as jnp
from jax.experimental import pallas as pl
from jax.experimental.pallas import tpu as pltpu

_BLOCK = 2048


def _tc_body(cnt_ref, bf_ref, pos_ref, out_ref):
    nseg = cnt_ref.shape[1]
    block = bf_ref.shape[0]
    i = pl.program_id(0)
    base = i * block

    cnt_f = cnt_ref[...].astype(jnp.float32)
    kk = jax.lax.broadcasted_iota(jnp.int32, (nseg, nseg), 0)
    jj = jax.lax.broadcasted_iota(jnp.int32, (nseg, nseg), 1)
    tri = (kk <= jj).astype(jnp.float32)
    ends = jax.lax.dot(cnt_f, tri, precision=jax.lax.Precision.HIGHEST)
    starts = ends - cnt_f

    r = (base + jax.lax.broadcasted_iota(jnp.int32, (block, 1), 0)
         ).astype(jnp.float32)
    onehot = ((r >= starts) & (r < ends)).astype(jnp.float32)
    inv = 1.0 / jnp.maximum(cnt_f, 1.0)
    scale = jnp.sum(onehot * inv, axis=1, keepdims=True)
    valid = jnp.sum(onehot, axis=1, keepdims=True) > 0.0
    scale = jnp.where(valid, scale, 1.0)
    posmat = jax.lax.dot(onehot, pos_ref[...],
                         precision=jax.lax.Precision.HIGHEST)
    out_ref[...] = bf_ref[...] * scale + posmat


def kernel(eachimg_selected_box_nums, box_features, positions):
    n, d = box_features.shape
    nseg = eachimg_selected_box_nums.shape[0]
    cnt = eachimg_selected_box_nums.astype(jnp.int32).reshape(1, nseg)
    grid = (n // _BLOCK,)
    return pl.pallas_call(
        _tc_body,
        grid=grid,
        in_specs=[
            pl.BlockSpec((1, nseg), lambda i: (0, 0)),
            pl.BlockSpec((_BLOCK, d), lambda i: (i, 0)),
            pl.BlockSpec((nseg, d), lambda i: (0, 0)),
        ],
        out_specs=pl.BlockSpec((_BLOCK, d), lambda i: (i, 0)),
        out_shape=jax.ShapeDtypeStruct((n, d), jnp.float32),
        compiler_params=pltpu.CompilerParams(
            dimension_semantics=("arbitrary",),
        ),
    )(cnt, box_features, positions)
